# baseline (speedup 1.0000x reference)
import functools

import jax
import jax.numpy as jnp
from jax import lax
from jax.experimental import pallas as pl
from jax.experimental.pallas import tpu as pltpu, tpu_sc as plsc

NC = 2
NS = 16
L = 16
NW = NC * NS

B = 16384
D = 128
BPW = B // NW
CHUNK = 128
NCHUNK = BPW // CHUNK
NBUF = 3


def _dots_body(users_hbm, items_hbm, uf_hbm, if_hbm, out_hbm,
               fidx_u, fidx_i, u_bufs, v_bufs, out_v, sem_f):
    wid = lax.axis_index("s") * NC + lax.axis_index("c")
    base = pl.multiple_of(wid * BPW, BPW)

    pltpu.sync_copy(users_hbm.at[pl.ds(base, BPW)], fidx_u)
    pltpu.sync_copy(items_hbm.at[pl.ds(base, BPW)], fidx_i)

    descs = {}

    def issue(c):
        slot = c % NBUF
        descs[(c, 0)] = pltpu.async_copy(
            uf_hbm.at[fidx_u.at[pl.ds(c * CHUNK, CHUNK)]], u_bufs[slot],
            sem_f)
        descs[(c, 1)] = pltpu.async_copy(
            if_hbm.at[fidx_i.at[pl.ds(c * CHUNK, CHUNK)]], v_bufs[slot],
            sem_f)

    for c in range(NBUF):
        issue(c)

    lane = lax.iota(jnp.int32, L)

    for c in range(NCHUNK):
        slot = c % NBUF
        descs[(c, 0)].wait()
        descs[(c, 1)].wait()
        u_buf = u_bufs[slot]
        v_buf = v_bufs[slot]

        for g in range(CHUNK // L):
            rows = g * L + lane

            def d_body(i, acc, rows=rows, u_buf=u_buf, v_buf=v_buf):
                for k in range(4):
                    dd = jnp.full((L,), 0, jnp.int32) + (i * 4 + k)
                    u_d = plsc.load_gather(u_buf, [rows, dd])
                    v_d = plsc.load_gather(v_buf, [rows, dd])
                    acc = acc + u_d * v_d
                return acc

            acc = lax.fori_loop(0, D // 4, d_body,
                                jnp.zeros((L,), jnp.float32))
            out_v[pl.ds(c * CHUNK + g * L, L)] = acc

        if c + NBUF < NCHUNK:
            issue(c + NBUF)

    pltpu.sync_copy(out_v, out_hbm.at[pl.ds(base, BPW)])


def _bias_body(users_hbm, items_hbm, ub_hbm, ib_hbm, out_hbm,
               fidx_u, fidx_i, ub_v, ib_v, out_v, sem_b):
    wid = lax.axis_index("s") * NC + lax.axis_index("c")
    base = pl.multiple_of(wid * BPW, BPW)

    pltpu.sync_copy(users_hbm.at[pl.ds(base, BPW)], fidx_u)
    pltpu.sync_copy(items_hbm.at[pl.ds(base, BPW)], fidx_i)
    d_ub = pltpu.async_copy(ub_hbm.at[fidx_u], ub_v, sem_b)
    d_ib = pltpu.async_copy(ib_hbm.at[fidx_i], ib_v, sem_b)
    d_ub.wait()
    d_ib.wait()
    for g in range(BPW // L):
        off = g * L
        out_v[pl.ds(off, L)] = ub_v[pl.ds(off, L)] + ib_v[pl.ds(off, L)]
    pltpu.sync_copy(out_v, out_hbm.at[pl.ds(base, BPW)])


@functools.partial(jax.jit, static_argnames=())
def kernel(users, items, user_factors, item_factors, user_bias, item_bias,
           global_bias):
    mesh = plsc.VectorSubcoreMesh(core_axis_name="c", subcore_axis_name="s")
    run_dots = pl.kernel(
        _dots_body,
        out_type=jax.ShapeDtypeStruct((B,), jnp.float32),
        mesh=mesh,
        compiler_params=pltpu.CompilerParams(needs_layout_passes=False),
        scratch_types=[
            pltpu.VMEM((BPW,), jnp.int32),
            pltpu.VMEM((BPW,), jnp.int32),
            [pltpu.VMEM((CHUNK, D), jnp.float32)] * NBUF,
            [pltpu.VMEM((CHUNK, D), jnp.float32)] * NBUF,
            pltpu.VMEM((BPW,), jnp.float32),
            pltpu.SemaphoreType.DMA,
        ],
    )
    run_bias = pl.kernel(
        _bias_body,
        out_type=jax.ShapeDtypeStruct((B,), jnp.float32),
        mesh=mesh,
        compiler_params=pltpu.CompilerParams(needs_layout_passes=False),
        scratch_types=[
            pltpu.VMEM((BPW,), jnp.int32),
            pltpu.VMEM((BPW,), jnp.int32),
            pltpu.VMEM((BPW,), jnp.float32),
            pltpu.VMEM((BPW,), jnp.float32),
            pltpu.VMEM((BPW,), jnp.float32),
            pltpu.SemaphoreType.DMA,
        ],
    )
    dots = run_dots(users, items, user_factors, item_factors)
    ub = user_bias.reshape(-1)
    ib = item_bias.reshape(-1)
    bias = run_bias(users, items, ub, ib)
    return dots + bias + global_bias[0]


# speedup vs baseline: 1.6602x; 1.6602x over previous
import functools

import jax
import jax.numpy as jnp
from jax import lax
from jax.experimental import pallas as pl
from jax.experimental.pallas import tpu as pltpu, tpu_sc as plsc

NC = 2
NS = 16
L = 16
NW = NC * NS

B = 16384
D = 128
BPW = B // NW
CHUNK = 128
NCHUNK = BPW // CHUNK
NBUF = 3


def _dots_body(users_hbm, items_hbm, uf_hbm, if_hbm, out_hbm,
               fidx_u, fidx_i, u_bufs, v_bufs, out_v, sem_f):
    wid = lax.axis_index("s") * NC + lax.axis_index("c")
    base = pl.multiple_of(wid * BPW, BPW)

    pltpu.sync_copy(users_hbm.at[pl.ds(base, BPW)], fidx_u)
    pltpu.sync_copy(items_hbm.at[pl.ds(base, BPW)], fidx_i)

    descs = {}

    def issue(c):
        slot = c % NBUF
        descs[(c, 0)] = pltpu.async_copy(
            uf_hbm.at[fidx_u.at[pl.ds(c * CHUNK, CHUNK)]], u_bufs[slot],
            sem_f)
        descs[(c, 1)] = pltpu.async_copy(
            if_hbm.at[fidx_i.at[pl.ds(c * CHUNK, CHUNK)]], v_bufs[slot],
            sem_f)

    for c in range(NBUF):
        issue(c)

    lane = lax.iota(jnp.int32, L)

    for c in range(NCHUNK):
        slot = c % NBUF
        descs[(c, 0)].wait()
        descs[(c, 1)].wait()
        u_buf = u_bufs[slot]
        v_buf = v_bufs[slot]

        for g in range(CHUNK // L):
            rows = g * L + lane

            def d_body(i, acc, rows=rows, u_buf=u_buf, v_buf=v_buf):
                for k in range(4):
                    dd = jnp.full((L,), 0, jnp.int32) + (i * 4 + k)
                    u_d = plsc.load_gather(u_buf, [rows, dd])
                    v_d = plsc.load_gather(v_buf, [rows, dd])
                    acc = acc + u_d * v_d
                return acc

            acc = lax.fori_loop(0, D // 4, d_body,
                                jnp.zeros((L,), jnp.float32))
            out_v[pl.ds(c * CHUNK + g * L, L)] = acc

        if c + NBUF < NCHUNK:
            issue(c + NBUF)

    pltpu.sync_copy(out_v, out_hbm.at[pl.ds(base, BPW)])


def _bias_body(users_hbm, items_hbm, ub_hbm, ib_hbm, gb_hbm, dots_hbm,
               out_hbm, fidx_u, fidx_i, ub_v, ib_v, gb_v, dots_v, out_v,
               sem_b):
    wid = lax.axis_index("s") * NC + lax.axis_index("c")
    base = pl.multiple_of(wid * BPW, BPW)

    pltpu.sync_copy(users_hbm.at[pl.ds(base, BPW)], fidx_u)
    pltpu.sync_copy(items_hbm.at[pl.ds(base, BPW)], fidx_i)
    d_ub = pltpu.async_copy(ub_hbm.at[fidx_u], ub_v, sem_b)
    d_ib = pltpu.async_copy(ib_hbm.at[fidx_i], ib_v, sem_b)
    pltpu.sync_copy(dots_hbm.at[pl.ds(base, BPW)], dots_v)
    pltpu.sync_copy(gb_hbm, gb_v)
    gb16 = plsc.load_gather(gb_v, [jnp.zeros((L,), jnp.int32)])
    d_ub.wait()
    d_ib.wait()
    for g in range(BPW // L):
        off = g * L
        out_v[pl.ds(off, L)] = (dots_v[pl.ds(off, L)] + ub_v[pl.ds(off, L)]
                                + ib_v[pl.ds(off, L)] + gb16)
    pltpu.sync_copy(out_v, out_hbm.at[pl.ds(base, BPW)])


@functools.partial(jax.jit, static_argnames=())
def kernel(users, items, user_factors, item_factors, user_bias, item_bias,
           global_bias):
    mesh = plsc.VectorSubcoreMesh(core_axis_name="c", subcore_axis_name="s")
    run_dots = pl.kernel(
        _dots_body,
        out_type=jax.ShapeDtypeStruct((B,), jnp.float32),
        mesh=mesh,
        compiler_params=pltpu.CompilerParams(needs_layout_passes=False),
        scratch_types=[
            pltpu.VMEM((BPW,), jnp.int32),
            pltpu.VMEM((BPW,), jnp.int32),
            [pltpu.VMEM((CHUNK, D), jnp.float32)] * NBUF,
            [pltpu.VMEM((CHUNK, D), jnp.float32)] * NBUF,
            pltpu.VMEM((BPW,), jnp.float32),
            pltpu.SemaphoreType.DMA,
        ],
    )
    run_bias = pl.kernel(
        _bias_body,
        out_type=jax.ShapeDtypeStruct((B,), jnp.float32),
        mesh=mesh,
        compiler_params=pltpu.CompilerParams(needs_layout_passes=False),
        scratch_types=[
            pltpu.VMEM((BPW,), jnp.int32),
            pltpu.VMEM((BPW,), jnp.int32),
            pltpu.VMEM((BPW,), jnp.float32),
            pltpu.VMEM((BPW,), jnp.float32),
            pltpu.VMEM((1,), jnp.float32),
            pltpu.VMEM((BPW,), jnp.float32),
            pltpu.VMEM((BPW,), jnp.float32),
            pltpu.SemaphoreType.DMA,
        ],
    )
    dots = run_dots(users, items, user_factors, item_factors)
    ub = user_bias.reshape(-1)
    ib = item_bias.reshape(-1)
    return run_bias(users, items, ub, ib, global_bias, dots)


# speedup vs baseline: 1.6748x; 1.0088x over previous
import functools

import jax
import jax.numpy as jnp
from jax import lax
from jax.experimental import pallas as pl
from jax.experimental.pallas import tpu as pltpu, tpu_sc as plsc

NC = 2
NS = 16
L = 16
NW = NC * NS

B = 16384
D = 128
BPW = B // NW
CHUNK = 128
NCHUNK = BPW // CHUNK
NBUF = 3


def _dots_body(users_hbm, items_hbm, uf_hbm, if_hbm, out_hbm,
               fidx_u, fidx_i, u_bufs, v_bufs, out_v, sem_f):
    wid = lax.axis_index("s") * NC + lax.axis_index("c")
    base = pl.multiple_of(wid * BPW, BPW)

    pltpu.sync_copy(users_hbm.at[pl.ds(base, BPW)], fidx_u)
    pltpu.sync_copy(items_hbm.at[pl.ds(base, BPW)], fidx_i)

    descs = {}

    def issue(c):
        slot = c % NBUF
        descs[(c, 0)] = pltpu.async_copy(
            uf_hbm.at[fidx_u.at[pl.ds(c * CHUNK, CHUNK)]], u_bufs[slot],
            sem_f)
        descs[(c, 1)] = pltpu.async_copy(
            if_hbm.at[fidx_i.at[pl.ds(c * CHUNK, CHUNK)]], v_bufs[slot],
            sem_f)

    for c in range(NBUF):
        issue(c)

    lane = lax.iota(jnp.int32, L)

    for c in range(NCHUNK):
        slot = c % NBUF
        descs[(c, 0)].wait()
        descs[(c, 1)].wait()
        u_buf = u_bufs[slot]
        v_buf = v_bufs[slot]

        for g in range(CHUNK // L):
            rows = g * L + lane

            def d_body(i, acc, rows=rows, u_buf=u_buf, v_buf=v_buf):
                for k in range(4):
                    dd = jnp.full((L,), 0, jnp.int32) + (i * 4 + k)
                    u_d = plsc.load_gather(u_buf, [rows, dd])
                    v_d = plsc.load_gather(v_buf, [rows, dd])
                    acc = acc + u_d * v_d
                return acc

            acc = lax.fori_loop(0, D // 4, d_body,
                                jnp.zeros((L,), jnp.float32))
            out_v[pl.ds(c * CHUNK + g * L, L)] = acc

        if c + NBUF < NCHUNK:
            issue(c + NBUF)

    pltpu.sync_copy(out_v, out_hbm.at[pl.ds(base, BPW)])


def _bias_body(users_hbm, items_hbm, ub_hbm, ib_hbm, gb_hbm, dots_hbm,
               out_hbm, fidx_u, fidx_i, ub_v, ib_v, gb_v, dots_v, out_v,
               sem_b):
    wid = lax.axis_index("s") * NC + lax.axis_index("c")
    base = pl.multiple_of(wid * BPW, BPW)

    d_iu = pltpu.async_copy(users_hbm.at[pl.ds(base, BPW)], fidx_u, sem_b)
    d_ii = pltpu.async_copy(items_hbm.at[pl.ds(base, BPW)], fidx_i, sem_b)
    d_dv = pltpu.async_copy(dots_hbm.at[pl.ds(base, BPW)], dots_v, sem_b)
    d_gb = pltpu.async_copy(gb_hbm, gb_v, sem_b)
    d_iu.wait()
    d_ii.wait()
    d_ub = pltpu.async_copy(ub_hbm.at[fidx_u], ub_v, sem_b)
    d_ib = pltpu.async_copy(ib_hbm.at[fidx_i], ib_v, sem_b)
    d_dv.wait()
    d_gb.wait()
    gb16 = plsc.load_gather(gb_v, [jnp.zeros((L,), jnp.int32)])
    d_ub.wait()
    d_ib.wait()
    for g in range(BPW // L):
        off = g * L
        out_v[pl.ds(off, L)] = (dots_v[pl.ds(off, L)] + ub_v[pl.ds(off, L)]
                                + ib_v[pl.ds(off, L)] + gb16)
    pltpu.sync_copy(out_v, out_hbm.at[pl.ds(base, BPW)])


@functools.partial(jax.jit, static_argnames=())
def kernel(users, items, user_factors, item_factors, user_bias, item_bias,
           global_bias):
    mesh = plsc.VectorSubcoreMesh(core_axis_name="c", subcore_axis_name="s")
    run_dots = pl.kernel(
        _dots_body,
        out_type=jax.ShapeDtypeStruct((B,), jnp.float32),
        mesh=mesh,
        compiler_params=pltpu.CompilerParams(needs_layout_passes=False),
        scratch_types=[
            pltpu.VMEM((BPW,), jnp.int32),
            pltpu.VMEM((BPW,), jnp.int32),
            [pltpu.VMEM((CHUNK, D), jnp.float32)] * NBUF,
            [pltpu.VMEM((CHUNK, D), jnp.float32)] * NBUF,
            pltpu.VMEM((BPW,), jnp.float32),
            pltpu.SemaphoreType.DMA,
        ],
    )
    run_bias = pl.kernel(
        _bias_body,
        out_type=jax.ShapeDtypeStruct((B,), jnp.float32),
        mesh=mesh,
        compiler_params=pltpu.CompilerParams(needs_layout_passes=False),
        scratch_types=[
            pltpu.VMEM((BPW,), jnp.int32),
            pltpu.VMEM((BPW,), jnp.int32),
            pltpu.VMEM((BPW,), jnp.float32),
            pltpu.VMEM((BPW,), jnp.float32),
            pltpu.VMEM((1,), jnp.float32),
            pltpu.VMEM((BPW,), jnp.float32),
            pltpu.VMEM((BPW,), jnp.float32),
            pltpu.SemaphoreType.DMA,
        ],
    )
    dots = run_dots(users, items, user_factors, item_factors)
    ub = user_bias.reshape(-1)
    ib = item_bias.reshape(-1)
    return run_bias(users, items, ub, ib, global_bias, dots)
